# baseline (speedup 1.0000x reference)
import jax
import jax.numpy as jnp
from jax import lax
from jax.experimental import pallas as pl
from jax.experimental.pallas import tpu as pltpu

_NUM_CLASSES = 1000
_B = 64
_NBUF = 4


def _body(x_ref, out_ref, bufs, sems):
    n0, n1 = x_ref.shape
    nchunk = n0 // _B
    iota = lax.broadcasted_iota(jnp.int32, (_B, n1, _NUM_CLASSES), 2)

    def compute(c, b):
        idx = x_ref[pl.ds(c * _B, _B), :]
        bufs[pl.ds(b, 1)] = (idx[:, :, None] == iota).astype(jnp.int32)[None]
        pltpu.make_async_copy(
            bufs.at[b], out_ref.at[pl.ds(c * _B, _B)], sems.at[b]
        ).start()

    for c in range(_NBUF):
        compute(c, c)

    def loop_body(c, carry):
        b = lax.rem(c, _NBUF)
        pltpu.make_async_copy(
            bufs.at[b], out_ref.at[pl.ds((c - _NBUF) * _B, _B)], sems.at[b]
        ).wait()
        compute(c, b)
        return carry

    lax.fori_loop(_NBUF, nchunk, loop_body, 0)

    for b in range(_NBUF):
        pltpu.make_async_copy(
            bufs.at[b], out_ref.at[pl.ds(0, _B)], sems.at[b]
        ).wait()


def kernel(x1):
    n0, n1 = x1.shape
    x1 = x1.astype(jnp.int32)
    out = pl.pallas_call(
        _body,
        in_specs=[pl.BlockSpec(memory_space=pltpu.VMEM)],
        out_specs=pl.BlockSpec(memory_space=pl.ANY),
        out_shape=jax.ShapeDtypeStruct((n0, n1, _NUM_CLASSES), jnp.int32),
        scratch_shapes=[
            pltpu.VMEM((_NBUF, _B, n1, _NUM_CLASSES), jnp.int32),
            pltpu.SemaphoreType.DMA((_NBUF,)),
        ],
    )(x1)
    return out
